# baseline (speedup 1.0000x reference)
import functools

import jax
import jax.numpy as jnp
from jax import lax
from jax.experimental import pallas as pl
from jax.experimental.pallas import tpu as pltpu
from jax.experimental.pallas import tpu_sc as plsc

_MAX_REL = 32
_NTAB = 2 * _MAX_REL + 1


def _sc_expand(table_hbm, out_hbm, tab_v, buf_v, *, seq_len, d_model,
               seg_rows):
    wid = lax.axis_index("c") * 16 + lax.axis_index("s")
    base = wid * seg_rows
    pltpu.sync_copy(table_hbm, tab_v)
    lo = seq_len - 1 - _MAX_REL
    for r in range(seg_rows):
        src = jnp.clip(base + r - lo, 0, _NTAB - 1) * d_model
        for v in range(d_model // 16):
            buf_v[pl.ds(r * d_model + v * 16, 16)] = (
                tab_v[pl.ds(src + v * 16, 16)])
    pltpu.sync_copy(
        buf_v, out_hbm.at[pl.ds(base * d_model, seg_rows * d_model)])


def _tc_stream(e_ref, out_ref, g_ref, *, seq_len, d_model, rows_per_blk):
    @pl.when(pl.program_id(0) == 0)
    def _build_g():
        for k in range(8):
            g_ref[k, :, :] = e_ref[k:k + 2 * seq_len, :]

    for r in range(rows_per_blk):
        q = seq_len - 8 * (pl.program_id(0) * (rows_per_blk // 8) + r // 8 + 1)
        q = pl.multiple_of(q, 8)
        out_ref[r, :, :] = g_ref[7 - r % 8, pl.ds(q, seq_len), :]


def kernel(x, table):
    seq_len = x.shape[1]
    d_model = table.shape[1]
    num_workers = 16
    e_rows = 2 * seq_len + num_workers
    seg_rows = e_rows // num_workers

    mesh = plsc.VectorSubcoreMesh(core_axis_name="c", subcore_axis_name="s",
                                  num_cores=1)
    sc_body = functools.partial(
        _sc_expand, seq_len=seq_len, d_model=d_model, seg_rows=seg_rows)
    e = pl.kernel(
        sc_body,
        mesh=mesh,
        out_type=jax.ShapeDtypeStruct((e_rows * d_model,), jnp.float32),
        scratch_types=[
            pltpu.VMEM((_NTAB * d_model,), jnp.float32),
            pltpu.VMEM((seg_rows * d_model,), jnp.float32),
        ],
    )(table.reshape(_NTAB * d_model))
    e = e.reshape(e_rows, d_model)

    rows_per_blk = 8
    tc_body = functools.partial(
        _tc_stream, seq_len=seq_len, d_model=d_model,
        rows_per_blk=rows_per_blk)
    rel = pl.pallas_call(
        tc_body,
        grid=(seq_len // rows_per_blk,),
        in_specs=[pl.BlockSpec((e_rows, d_model), lambda i: (0, 0))],
        out_specs=pl.BlockSpec((rows_per_blk, seq_len, d_model),
                               lambda i: (i, 0, 0)),
        out_shape=jax.ShapeDtypeStruct((seq_len, seq_len, d_model),
                                       jnp.float32),
        scratch_shapes=[pltpu.VMEM((8, 2 * seq_len, d_model), jnp.float32)],
    )(e)
    return (x, rel)


# speedup vs baseline: 1.0317x; 1.0317x over previous
import functools

import jax
import jax.numpy as jnp
from jax import lax
from jax.experimental import pallas as pl
from jax.experimental.pallas import tpu as pltpu
from jax.experimental.pallas import tpu_sc as plsc

_MAX_REL = 32
_NTAB = 2 * _MAX_REL + 1


def _sc_expand(table_hbm, out_hbm, tab_v, buf_v, *, seq_len, d_model,
               seg_rows):
    wid = lax.axis_index("c") * 16 + lax.axis_index("s")
    base = wid * seg_rows
    lo = seq_len - 1 - _MAX_REL
    win = seg_rows + 1
    w0 = jnp.clip(base - lo, 0, _NTAB - win)
    pltpu.sync_copy(table_hbm.at[pl.ds(w0 * d_model, win * d_model)], tab_v)
    for r in range(seg_rows):
        src = (jnp.clip(base + r - lo, 0, _NTAB - 1) - w0) * d_model
        for v in range(d_model // 16):
            buf_v[pl.ds(r * d_model + v * 16, 16)] = (
                tab_v[pl.ds(src + v * 16, 16)])
    pltpu.sync_copy(
        buf_v, out_hbm.at[pl.ds(base * d_model, seg_rows * d_model)])


def _tc_stream(e_ref, out_ref, g_ref, *, seq_len, d_model, rows_per_blk):
    @pl.when(pl.program_id(0) == 0)
    def _build_g():
        for k in range(8):
            g_ref[k, :, :] = e_ref[k:k + 2 * seq_len, :]

    for r in range(rows_per_blk):
        q = seq_len - 8 * (pl.program_id(0) * (rows_per_blk // 8) + r // 8 + 1)
        q = pl.multiple_of(q, 8)
        out_ref[r, :, :] = g_ref[7 - r % 8, pl.ds(q, seq_len), :]


def kernel(x, table):
    seq_len = x.shape[1]
    d_model = table.shape[1]
    num_workers = 32
    e_rows = 2 * seq_len + num_workers
    seg_rows = e_rows // num_workers

    mesh = plsc.VectorSubcoreMesh(core_axis_name="c", subcore_axis_name="s")
    sc_body = functools.partial(
        _sc_expand, seq_len=seq_len, d_model=d_model, seg_rows=seg_rows)
    e = pl.kernel(
        sc_body,
        mesh=mesh,
        out_type=jax.ShapeDtypeStruct((e_rows * d_model,), jnp.float32),
        scratch_types=[
            pltpu.VMEM(((seg_rows + 1) * d_model,), jnp.float32),
            pltpu.VMEM((seg_rows * d_model,), jnp.float32),
        ],
    )(table.reshape(_NTAB * d_model))
    e = e.reshape(e_rows, d_model)

    rows_per_blk = 8
    tc_body = functools.partial(
        _tc_stream, seq_len=seq_len, d_model=d_model,
        rows_per_blk=rows_per_blk)
    rel = pl.pallas_call(
        tc_body,
        grid=(seq_len // rows_per_blk,),
        in_specs=[pl.BlockSpec((e_rows, d_model), lambda i: (0, 0))],
        out_specs=pl.BlockSpec((rows_per_blk, seq_len, d_model),
                               lambda i: (i, 0, 0)),
        out_shape=jax.ShapeDtypeStruct((seq_len, seq_len, d_model),
                                       jnp.float32),
        scratch_shapes=[pltpu.VMEM((8, 2 * seq_len, d_model), jnp.float32)],
    )(e)
    return (x, rel)
